# baseline (speedup 1.0000x reference)
import functools

import jax
import jax.numpy as jnp
from jax import lax
from jax.experimental import pallas as pl
from jax.experimental.pallas import tpu as pltpu
from jax.experimental.pallas import tpu_sc as plsc

T = 16384
DIM = 2048
NE = 64
TOPK = 8
NG = 8
TOPK_G = 4
SCALE = 2.5
GSZ = NE // NG

NEG = -1e30
L = 16
NW = 32
TW = T // NW
NB = TW // L


def _mm_body(x_ref, wt_ref, s_ref):
    s_ref[...] = jax.nn.sigmoid(
        jnp.dot(x_ref[...], wt_ref[...], preferred_element_type=jnp.float32))


def _tc_scores(x, wt, rows=2048):
    nt = x.shape[0]
    return pl.pallas_call(
        _mm_body,
        grid=(nt // rows,),
        in_specs=[
            pl.BlockSpec((rows, DIM), lambda i: (i, 0)),
            pl.BlockSpec((DIM, NE), lambda i: (0, 0)),
        ],
        out_specs=pl.BlockSpec((rows, NE), lambda i: (i, 0)),
        out_shape=jax.ShapeDtypeStruct((nt, NE), jnp.float32),
    )(x, wt)


def _splat(v, dtype=jnp.int32):
    return jnp.full((L,), v, dtype)


def _sc_body(tw, s_hbm, w_hbm, i_hbm, s_v, w_v, i_v):
    TW = tw
    NB = TW // L
    wid = lax.axis_index("s") * 2 + lax.axis_index("c")
    base = wid * TW
    pltpu.sync_copy(s_hbm.at[pl.ds(base * NE, TW * NE)], s_v)

    iota = lax.broadcasted_iota(jnp.int32, (L,), 0)

    def batch(b, carry):
        tokNE = (b * L + iota) * NE
        tokK = (b * L + iota) * TOPK

        gm = []
        for g in range(NG):
            vs = [plsc.load_gather(s_v, [tokNE + (g * GSZ + k)])
                  for k in range(GSZ)]
            while len(vs) > 1:
                vs = [jnp.maximum(vs[i], vs[i + 1])
                      for i in range(0, len(vs), 2)]
            gm.append(vs[0])

        gsel = []
        for _ in range(TOPK_G):
            t = gm
            while len(t) > 1:
                t = [jnp.maximum(t[i], t[i + 1]) for i in range(0, len(t), 2)]
            cur = t[0]
            gs = _splat(127)
            for g in range(NG - 1, -1, -1):
                gs = jnp.where(gm[g] == cur, g, gs)
            gsel.append(gs)
            gm = [jnp.where(gs == g, NEG, gm[g]) for g in range(NG)]

        a, b_, c, d = gsel
        lo0, hi0 = jnp.minimum(a, b_), jnp.maximum(a, b_)
        lo1, hi1 = jnp.minimum(c, d), jnp.maximum(c, d)
        g0 = jnp.minimum(lo0, lo1)
        t0 = jnp.maximum(lo0, lo1)
        g3 = jnp.maximum(hi0, hi1)
        t1 = jnp.minimum(hi0, hi1)
        g1 = jnp.minimum(t0, t1)
        g2 = jnp.maximum(t0, t1)
        bases = [g0 * GSZ, g1 * GSZ, g2 * GSZ, g3 * GSZ]

        cand = [plsc.load_gather(s_v, [tokNE + bases[q] + k])
                for q in range(TOPK_G) for k in range(GSZ)]

        wsum = None
        wouts, eouts = [], []
        for j in range(TOPK):
            tv = list(cand)
            te = [bases[i // GSZ] + (i % GSZ) for i in range(len(cand))]
            while len(tv) > 1:
                nv, ne_ = [], []
                for i in range(0, len(tv), 2):
                    better = tv[i + 1] > tv[i]
                    nv.append(jnp.where(better, tv[i + 1], tv[i]))
                    ne_.append(jnp.where(better, te[i + 1], te[i]))
                tv, te = nv, ne_
            cur, eb = tv[0], te[0]
            wouts.append(cur)
            eouts.append(eb)
            wsum = cur if wsum is None else wsum + cur
            dq = [eb - bases[q] for q in range(TOPK_G)]
            cand = [jnp.where(dq[i // GSZ] == (i % GSZ), NEG, cand[i])
                    for i in range(len(cand))]

        inv = SCALE / wsum
        for j in range(TOPK):
            plsc.store_scatter(w_v, [tokK + j], wouts[j] * inv)
            plsc.store_scatter(i_v, [tokK + j], eouts[j])
        return carry

    lax.fori_loop(0, NB, batch, 0)

    pltpu.sync_copy(w_v, w_hbm.at[pl.ds(base * TOPK, TW * TOPK)])
    pltpu.sync_copy(i_v, i_hbm.at[pl.ds(base * TOPK, TW * TOPK)])


def _sc_route(scores_flat):
    nt = scores_flat.shape[0] // NE
    tw = nt // NW
    mesh = plsc.VectorSubcoreMesh(core_axis_name="c", subcore_axis_name="s")
    call = functools.partial(
        pl.kernel,
        mesh=mesh,
        compiler_params=pltpu.CompilerParams(use_tc_tiling_on_sc=False,
                                             needs_layout_passes=False),
        out_type=[
            jax.ShapeDtypeStruct((nt * TOPK,), jnp.float32),
            jax.ShapeDtypeStruct((nt * TOPK,), jnp.int32),
        ],
        scratch_types=[
            pltpu.VMEM((tw * NE,), jnp.float32),
            pltpu.VMEM((tw * TOPK,), jnp.float32),
            pltpu.VMEM((tw * TOPK,), jnp.int32),
        ],
    )(functools.partial(_sc_body, tw))
    return call(scores_flat)


@functools.partial(jax.jit, static_argnames=("nchunks",))
def _route_all(x, wt, nchunks=4):
    ct = T // nchunks
    ws, is_ = [], []
    for c in range(nchunks):
        scores = _tc_scores(
            jax.lax.slice(x, (c * ct, 0), ((c + 1) * ct, DIM)), wt)
        w, i = _sc_route(scores.reshape(-1))
        ws.append(w.reshape(ct, TOPK))
        is_.append(i.reshape(ct, TOPK))
    return jnp.concatenate(ws, axis=0), jnp.concatenate(is_, axis=0)


def kernel(x, weight):
    return _route_all(x, weight.T)


# speedup vs baseline: 3.8372x; 3.8372x over previous
import functools

import jax
import jax.numpy as jnp
from jax.experimental import pallas as pl

T = 16384
DIM = 2048
NE = 64
TOPK = 8
NG = 8
TOPK_G = 4
SCALE = 2.5
GSZ = NE // NG

NEG = -1e30


def _router_body(x_ref, wt_ref, w_out_ref, i_out_ref):
    r = x_ref.shape[0]
    logits = jnp.dot(x_ref[...], wt_ref[...],
                     preferred_element_type=jnp.float32)
    st = jax.nn.sigmoid(logits.T)

    row = jax.lax.broadcasted_iota(jnp.int32, (NE, r), 0)
    grow = jax.lax.broadcasted_iota(jnp.int32, (NG, r), 0)

    gmax = jnp.concatenate(
        [jnp.max(st[g * GSZ:(g + 1) * GSZ, :], axis=0, keepdims=True)
         for g in range(NG)], axis=0)

    work = gmax
    keep = jnp.zeros((NE, r), jnp.bool_)
    for _ in range(TOPK_G):
        m = jnp.max(work, axis=0, keepdims=True)
        mg = jnp.min(jnp.where(work >= m, grow, 127),
                     axis=0, keepdims=True)
        keep = jnp.logical_or(keep, (row // GSZ) == mg)
        work = jnp.where(grow == mg, NEG, work)

    sm = jnp.where(keep, st, NEG)
    vals, idxs = [], []
    for _ in range(TOPK):
        m = jnp.max(sm, axis=0, keepdims=True)
        mi = jnp.min(jnp.where(sm >= m, row, 127),
                     axis=0, keepdims=True)
        vals.append(m)
        idxs.append(mi)
        sm = jnp.where(row == mi, NEG, sm)

    v = jnp.concatenate(vals, axis=0)
    w_out_ref[...] = v * (SCALE / jnp.sum(v, axis=0, keepdims=True))
    i_out_ref[...] = jnp.concatenate(idxs, axis=0)


@functools.partial(jax.jit, static_argnames=("rows",))
def _route(x, wt, rows=2048):
    grid = (T // rows,)
    return pl.pallas_call(
        _router_body,
        grid=grid,
        in_specs=[
            pl.BlockSpec((rows, DIM), lambda i: (i, 0)),
            pl.BlockSpec((DIM, NE), lambda i: (0, 0)),
        ],
        out_specs=[
            pl.BlockSpec((TOPK, rows), lambda i: (0, i)),
            pl.BlockSpec((TOPK, rows), lambda i: (0, i)),
        ],
        out_shape=[
            jax.ShapeDtypeStruct((TOPK, T), jnp.float32),
            jax.ShapeDtypeStruct((TOPK, T), jnp.int32),
        ],
    )(x, wt)


def kernel(x, weight):
    w, i = _route(x, weight.T)
    return w.T, i.T
